# baseline (speedup 1.0000x reference)
import jax
import jax.numpy as jnp
from jax import lax
from jax.experimental import pallas as pl
from jax.experimental.pallas import tpu as pltpu
from jax.experimental.pallas import tpu_sc as plsc

N = 10000
E = 320000
CH = 128
NCH = E // CH
NCHP = 2560
KPT = NCHP // 32
NW = 32
NP = 10240
RPS = NP // 16


def _make_seg_sum(width, with_cnt):
  mesh = plsc.VectorSubcoreMesh(core_axis_name="c", subcore_axis_name="s")
  out_type = [jax.ShapeDtypeStruct((2, NP, width), jnp.float32)]
  if with_cnt:
    out_type.append(jax.ShapeDtypeStruct((2, NP // CH, CH), jnp.float32))
  scratch = [
      pltpu.VMEM((2, CH), jnp.int32),
      pltpu.VMEM((2, CH), jnp.int32),
      pltpu.VMEM((CH, width), jnp.float32),
      pltpu.VMEM((CH, width), jnp.float32),
      pltpu.VMEM_SHARED((NP, width), jnp.float32),
      pltpu.SemaphoreType.DMA,
      pltpu.SemaphoreType.DMA,
  ]
  if with_cnt:
    scratch += [
        pltpu.VMEM((NP // CH, CH), jnp.float32),
        pltpu.VMEM((NP // CH,), jnp.int32),
        pltpu.VMEM_SHARED((NP // CH, CH), jnp.float32),
    ]
  NSL = NP // CH
  SPS = NSL // 16

  def body(*refs):
    if with_cnt:
      (p_hbm, ei_hbm, z2_hbm, z80_hbm, acc_hbm, cnt_hbm,
       idx0, idx1, rows0, rows1, acc_sh, sem0, sem1,
       cnt_priv, iota_v, cnt_sh) = refs
    else:
      (p_hbm, ei_hbm, z2_hbm, acc_hbm,
       idx0, idx1, rows0, rows1, acc_sh, sem0, sem1) = refs
    c = lax.axis_index("c")
    s = lax.axis_index("s")
    wid = s * 2 + c
    pltpu.sync_copy(z2_hbm.at[pl.ds(s * RPS, RPS)],
                    acc_sh.at[pl.ds(s * RPS, RPS)])
    if with_cnt:
      pltpu.sync_copy(z80_hbm.at[pl.ds(s * SPS, SPS)],
                      cnt_sh.at[pl.ds(s * SPS, SPS)])
      pltpu.sync_copy(z80_hbm, cnt_priv)
      for j in range(NSL // 16):
        iota_v[pl.ds(j * 16, 16)] = lax.iota(jnp.int32, 16) + 16 * j
    plsc.subcore_barrier()
    ones16 = jnp.ones((16,), jnp.float32)

    def load_fire(j, idx, rows, sem):
      @pl.when(j < NCH)
      def _():
        pltpu.sync_copy(ei_hbm.at[0, pl.ds(j * CH, CH)], idx.at[0])
        pltpu.sync_copy(ei_hbm.at[1, pl.ds(j * CH, CH)], idx.at[1])
        pltpu.async_copy(p_hbm.at[idx.at[0]], rows, sem)

    def drain_scatter(j, idx, rows, sem):
      @pl.when(j < NCH)
      def _():
        pltpu.make_async_copy(p_hbm.at[idx.at[0]], rows, sem).wait()
        pltpu.sync_copy(rows, acc_sh.at[idx.at[1]], add=True)
        if with_cnt:
          for cc in range(CH // 16):
            d = idx[1, pl.ds(cc * 16, 16)]
            plsc.addupdate_scatter(
                cnt_priv, [lax.shift_right_logical(d, 7),
                           lax.bitwise_and(d, 127)], ones16)

    load_fire(wid, idx0, rows0, sem0)

    @pl.loop(0, KPT // 2 - 1)
    def _(i):
      base = wid + 2 * i * NW
      load_fire(base + NW, idx1, rows1, sem1)
      drain_scatter(base, idx0, rows0, sem0)
      load_fire(base + 2 * NW, idx0, rows0, sem0)
      drain_scatter(base + NW, idx1, rows1, sem1)

    load_fire(wid + (KPT - 1) * NW, idx1, rows1, sem1)
    drain_scatter(wid + (KPT - 2) * NW, idx0, rows0, sem0)
    drain_scatter(wid + (KPT - 1) * NW, idx1, rows1, sem1)

    if with_cnt:
      pltpu.sync_copy(cnt_priv, cnt_sh.at[iota_v], add=True)
    plsc.subcore_barrier()
    pltpu.sync_copy(acc_sh.at[pl.ds(s * RPS, RPS)],
                    acc_hbm.at[c, pl.ds(s * RPS, RPS)])
    if with_cnt:
      pltpu.sync_copy(cnt_sh.at[pl.ds(s * SPS, SPS)],
                      cnt_hbm.at[c, pl.ds(s * SPS, SPS)])

  return pl.kernel(
      body, out_type=out_type, mesh=mesh, scratch_types=scratch,
      compiler_params=pltpu.CompilerParams(
          use_tc_tiling_on_sc=False,
          needs_layout_passes=not with_cnt))


_seg_sum_cnt_64 = _make_seg_sum(64, True)
_seg_sum_32 = _make_seg_sum(32, False)


def _mm_body(x_ref, w_ref, o1_ref, o2_ref):
  xw = jnp.dot(x_ref[...], w_ref[...], preferred_element_type=jnp.float32)
  h = xw.shape[1] // 2
  o1_ref[...] = xw[:, :h]
  o2_ref[...] = xw[:, h:]


def _proj(x, w, bm):
  m, k = x.shape
  n = w.shape[1]
  return pl.pallas_call(
      _mm_body,
      out_shape=[jax.ShapeDtypeStruct((m, n // 2), jnp.float32),
                 jax.ShapeDtypeStruct((m, n // 2), jnp.float32)],
      grid=(m // bm,),
      in_specs=[
          pl.BlockSpec((bm, k), lambda i: (i, 0)),
          pl.BlockSpec((k, n), lambda i: (0, 0)),
      ],
      out_specs=[pl.BlockSpec((bm, n // 2), lambda i: (i, 0)),
                 pl.BlockSpec((bm, n // 2), lambda i: (i, 0))],
  )(x, w)


def _combine_body(a0, a1, c0, c1, r, b, w, o1, o2):
  cnt = jnp.maximum(c0[...] + c1[...], 1.0)
  h = (a0[...] + a1[...]) / cnt + b[...] + r[...]
  h = jnp.maximum(h, 0.0)
  hw = jnp.dot(h, w[...], preferred_element_type=jnp.float32)
  n2 = hw.shape[1] // 2
  o1_ref = hw[:, :n2]
  o1[...] = o1_ref
  o2[...] = hw[:, n2:]


def _combine(acc, c0, c1, r, b, w, bm):
  m, d = r.shape
  n = w.shape[1]
  return pl.pallas_call(
      _combine_body,
      out_shape=[jax.ShapeDtypeStruct((m, n // 2), jnp.float32),
                 jax.ShapeDtypeStruct((m, n // 2), jnp.float32)],
      grid=(m // bm,),
      in_specs=[
          pl.BlockSpec((None, bm, d), lambda i: (0, i, 0)),
          pl.BlockSpec((None, bm, d), lambda i: (1, i, 0)),
          pl.BlockSpec((bm, 1), lambda i: (i, 0)),
          pl.BlockSpec((bm, 1), lambda i: (i, 0)),
          pl.BlockSpec((bm, d), lambda i: (i, 0)),
          pl.BlockSpec((1, d), lambda i: (0, 0)),
          pl.BlockSpec((d, n), lambda i: (0, 0)),
      ],
      out_specs=[pl.BlockSpec((bm, n // 2), lambda i: (i, 0)),
                 pl.BlockSpec((bm, n // 2), lambda i: (i, 0))],
  )(acc, acc, c0, c1, r, b, w)


def _head_body(a0, a1, c0, c1, r, b, w, bh, o):
  cnt = jnp.maximum(c0[...] + c1[...], 1.0)
  h = (a0[...] + a1[...]) / cnt + b[...] + r[...]
  h = jnp.maximum(h, 0.0)
  o[...] = jnp.dot(h, w[...], preferred_element_type=jnp.float32) + bh[...]


def _head(acc, c0, c1, r, b, w, bh, bm):
  m, d = r.shape
  return pl.pallas_call(
      _head_body,
      out_shape=jax.ShapeDtypeStruct((m, 1), jnp.float32),
      grid=(m // bm,),
      in_specs=[
          pl.BlockSpec((None, bm, d), lambda i: (0, i, 0)),
          pl.BlockSpec((None, bm, d), lambda i: (1, i, 0)),
          pl.BlockSpec((bm, 1), lambda i: (i, 0)),
          pl.BlockSpec((bm, 1), lambda i: (i, 0)),
          pl.BlockSpec((bm, d), lambda i: (i, 0)),
          pl.BlockSpec((1, d), lambda i: (0, 0)),
          pl.BlockSpec((d, 1), lambda i: (0, 0)),
          pl.BlockSpec((1, 1), lambda i: (0, 0)),
      ],
      out_specs=pl.BlockSpec((bm, 1), lambda i: (i, 0)),
  )(acc, acc, c0, c1, r, b, w, bh)


@jax.jit
def kernel(x, ei, Wl1, bl1, Wr1, Wl2, bl2, Wr2, Wh, bh):
  eij = ei.astype(jnp.int32)

  w1 = jnp.concatenate([Wl1.T, Wr1.T], axis=1)
  p1, r1 = _proj(x, w1, 2000)

  z2 = jnp.zeros((NP, 64), jnp.float32)
  z80 = jnp.zeros((NP // CH, CH), jnp.float32)
  acc1, cnt = _seg_sum_cnt_64(p1, eij, z2, z80)
  cnt = cnt.reshape(2, NP)
  c0 = cnt[0, :N].reshape(N, 1)
  c1 = cnt[1, :N].reshape(N, 1)

  w2 = jnp.concatenate([Wl2.T, Wr2.T], axis=1)
  p2, r2 = _combine(acc1, c0, c1, r1,
                    bl1.reshape(1, 64), w2, 2000)

  z32 = jnp.zeros((NP, 32), jnp.float32)
  (acc2,) = _seg_sum_32(p2, eij, z32)

  out = _head(acc2, c0, c1, r2,
              bl2.reshape(1, 32), Wh.T, bh.reshape(1, 1), 2000)
  return out.reshape(N)


# speedup vs baseline: 1.2250x; 1.2250x over previous
import jax
import jax.numpy as jnp
from jax import lax
from jax.experimental import pallas as pl
from jax.experimental.pallas import tpu as pltpu
from jax.experimental.pallas import tpu_sc as plsc

N = 10000
E = 320000
CH = 128
NCH = E // CH
NCHP = 2560
KPT = NCHP // 32
NW = 32
NP = 10240
RPS = NP // 16


def _make_seg_sum(width, with_cnt):
  mesh = plsc.VectorSubcoreMesh(core_axis_name="c", subcore_axis_name="s")
  out_type = [jax.ShapeDtypeStruct((2, NP, width), jnp.float32)]
  if with_cnt:
    out_type.append(jax.ShapeDtypeStruct((2, NP), jnp.float32))
  scratch = [
      pltpu.VMEM((2, CH), jnp.int32),
      pltpu.VMEM((2, CH), jnp.int32),
      pltpu.VMEM((CH, width), jnp.float32),
      pltpu.VMEM((CH, width), jnp.float32),
      pltpu.VMEM_SHARED((NP, width), jnp.float32),
      pltpu.SemaphoreType.DMA,
      pltpu.SemaphoreType.DMA,
  ]
  if with_cnt:
    scratch += [
        pltpu.VMEM((CH,), jnp.float32),
        pltpu.VMEM_SHARED((NP,), jnp.float32),
    ]

  def body(*refs):
    if with_cnt:
      (p_hbm, eij_hbm, z2_hbm, z1_hbm, acc_hbm, cnt_hbm,
       idx0, idx1, rows0, rows1, acc_sh, sem0, sem1, ones_v, cnt_sh) = refs
    else:
      (p_hbm, eij_hbm, z2_hbm, acc_hbm,
       idx0, idx1, rows0, rows1, acc_sh, sem0, sem1) = refs
    c = lax.axis_index("c")
    s = lax.axis_index("s")
    wid = s * 2 + c
    pltpu.sync_copy(z2_hbm.at[pl.ds(s * RPS, RPS)],
                    acc_sh.at[pl.ds(s * RPS, RPS)])
    if with_cnt:
      pltpu.sync_copy(z1_hbm.at[pl.ds(s * RPS, RPS)],
                      cnt_sh.at[pl.ds(s * RPS, RPS)])
      for j in range(CH // 16):
        ones_v[pl.ds(j * 16, 16)] = jnp.ones((16,), jnp.float32)
    plsc.subcore_barrier()

    def load_fire(j, idx, rows, sem):
      pltpu.sync_copy(eij_hbm.at[j], idx)
      pltpu.async_copy(p_hbm.at[idx.at[0]], rows, sem)

    def drain_scatter(idx, rows, sem):
      pltpu.make_async_copy(p_hbm.at[idx.at[0]], rows, sem).wait()
      pltpu.sync_copy(rows, acc_sh.at[idx.at[1]], add=True)
      if with_cnt:
        pltpu.sync_copy(ones_v, cnt_sh.at[idx.at[1]], add=True)

    load_fire(wid, idx0, rows0, sem0)

    @pl.loop(0, KPT // 2 - 1)
    def _(i):
      base = wid + 2 * i * NW
      load_fire(base + NW, idx1, rows1, sem1)
      drain_scatter(idx0, rows0, sem0)
      load_fire(base + 2 * NW, idx0, rows0, sem0)
      drain_scatter(idx1, rows1, sem1)

    load_fire(wid + (KPT - 1) * NW, idx1, rows1, sem1)
    drain_scatter(idx0, rows0, sem0)
    drain_scatter(idx1, rows1, sem1)

    plsc.subcore_barrier()
    pltpu.sync_copy(acc_sh.at[pl.ds(s * RPS, RPS)],
                    acc_hbm.at[c, pl.ds(s * RPS, RPS)])
    if with_cnt:
      pltpu.sync_copy(cnt_sh.at[pl.ds(s * RPS, RPS)],
                      cnt_hbm.at[c, pl.ds(s * RPS, RPS)])

  return pl.kernel(
      body, out_type=out_type, mesh=mesh, scratch_types=scratch,
      compiler_params=pltpu.CompilerParams(use_tc_tiling_on_sc=False))


_seg_sum_cnt_64 = _make_seg_sum(64, True)
_seg_sum_32 = _make_seg_sum(32, False)


def _mm_body(x_ref, w_ref, o1_ref, o2_ref):
  xw = jnp.dot(x_ref[...], w_ref[...], preferred_element_type=jnp.float32)
  h = xw.shape[1] // 2
  o1_ref[...] = xw[:, :h]
  o2_ref[...] = xw[:, h:]


def _proj(x, w, bm):
  m, k = x.shape
  n = w.shape[1]
  return pl.pallas_call(
      _mm_body,
      out_shape=[jax.ShapeDtypeStruct((m, n // 2), jnp.float32),
                 jax.ShapeDtypeStruct((m, n // 2), jnp.float32)],
      grid=(m // bm,),
      in_specs=[
          pl.BlockSpec((bm, k), lambda i: (i, 0)),
          pl.BlockSpec((k, n), lambda i: (0, 0)),
      ],
      out_specs=[pl.BlockSpec((bm, n // 2), lambda i: (i, 0)),
                 pl.BlockSpec((bm, n // 2), lambda i: (i, 0))],
  )(x, w)


def _combine_body(a0, a1, c0, c1, r, b, w, o1, o2):
  cnt = jnp.maximum(c0[...] + c1[...], 1.0)
  h = (a0[...] + a1[...]) / cnt + b[...] + r[...]
  h = jnp.maximum(h, 0.0)
  hw = jnp.dot(h, w[...], preferred_element_type=jnp.float32)
  n2 = hw.shape[1] // 2
  o1_ref = hw[:, :n2]
  o1[...] = o1_ref
  o2[...] = hw[:, n2:]


def _combine(acc, c0, c1, r, b, w, bm):
  m, d = r.shape
  n = w.shape[1]
  return pl.pallas_call(
      _combine_body,
      out_shape=[jax.ShapeDtypeStruct((m, n // 2), jnp.float32),
                 jax.ShapeDtypeStruct((m, n // 2), jnp.float32)],
      grid=(m // bm,),
      in_specs=[
          pl.BlockSpec((None, bm, d), lambda i: (0, i, 0)),
          pl.BlockSpec((None, bm, d), lambda i: (1, i, 0)),
          pl.BlockSpec((bm, 1), lambda i: (i, 0)),
          pl.BlockSpec((bm, 1), lambda i: (i, 0)),
          pl.BlockSpec((bm, d), lambda i: (i, 0)),
          pl.BlockSpec((1, d), lambda i: (0, 0)),
          pl.BlockSpec((d, n), lambda i: (0, 0)),
      ],
      out_specs=[pl.BlockSpec((bm, n // 2), lambda i: (i, 0)),
                 pl.BlockSpec((bm, n // 2), lambda i: (i, 0))],
  )(acc, acc, c0, c1, r, b, w)


def _head_body(a0, a1, c0, c1, r, b, w, bh, o):
  cnt = jnp.maximum(c0[...] + c1[...], 1.0)
  h = (a0[...] + a1[...]) / cnt + b[...] + r[...]
  h = jnp.maximum(h, 0.0)
  o[...] = jnp.dot(h, w[...], preferred_element_type=jnp.float32) + bh[...]


def _head(acc, c0, c1, r, b, w, bh, bm):
  m, d = r.shape
  return pl.pallas_call(
      _head_body,
      out_shape=jax.ShapeDtypeStruct((m, 1), jnp.float32),
      grid=(m // bm,),
      in_specs=[
          pl.BlockSpec((None, bm, d), lambda i: (0, i, 0)),
          pl.BlockSpec((None, bm, d), lambda i: (1, i, 0)),
          pl.BlockSpec((bm, 1), lambda i: (i, 0)),
          pl.BlockSpec((bm, 1), lambda i: (i, 0)),
          pl.BlockSpec((bm, d), lambda i: (i, 0)),
          pl.BlockSpec((1, d), lambda i: (0, 0)),
          pl.BlockSpec((d, 1), lambda i: (0, 0)),
          pl.BlockSpec((1, 1), lambda i: (0, 0)),
      ],
      out_specs=pl.BlockSpec((bm, 1), lambda i: (i, 0)),
  )(acc, acc, c0, c1, r, b, w, bh)


@jax.jit
def kernel(x, ei, Wl1, bl1, Wr1, Wl2, bl2, Wr2, Wh, bh):
  eij = ei.astype(jnp.int32).reshape(2, NCH, CH).transpose(1, 0, 2)
  lanes = lax.iota(jnp.int32, CH)
  pad = jnp.broadcast_to(
      jnp.stack([lanes, lanes + N], axis=0)[None], (NCHP - NCH, 2, CH))
  eij = jnp.concatenate([eij, pad], axis=0)

  w1 = jnp.concatenate([Wl1.T, Wr1.T], axis=1)
  p1, r1 = _proj(x, w1, 2000)

  z2 = jnp.zeros((NP, 64), jnp.float32)
  z1 = jnp.zeros((NP,), jnp.float32)
  acc1, cnt = _seg_sum_cnt_64(p1, eij, z2, z1)
  c0 = cnt[0, :N].reshape(N, 1)
  c1 = cnt[1, :N].reshape(N, 1)

  w2 = jnp.concatenate([Wl2.T, Wr2.T], axis=1)
  p2, r2 = _combine(acc1, c0, c1, r1,
                    bl1.reshape(1, 64), w2, 2000)

  z32 = jnp.zeros((NP, 32), jnp.float32)
  (acc2,) = _seg_sum_32(p2, eij, z32)

  out = _head(acc2, c0, c1, r2,
              bl2.reshape(1, 32), Wh.T, bh.reshape(1, 1), 2000)
  return out.reshape(N)
